# baseline (speedup 1.0000x reference)
import math

import numpy as np
import jax
import jax.numpy as jnp
from jax.experimental import pallas as pl
from jax.experimental.pallas import tpu as pltpu

_L = 2048
_KH = 1152
_n = np.arange(_L, dtype=np.int64)
_ang = (2.0 * np.pi / _L) * (np.outer(_n, _n) % _L)
_COS_NP = np.cos(_ang).astype(np.float32)
_SIN_NP = np.sin(_ang).astype(np.float32)
_FC_NP = _COS_NP[:, :_KH].copy()
_FS_NP = _SIN_NP[:, :_KH].copy()
_WGT_NP = np.zeros((1, _KH), np.float32)
_WGT_NP[0, 0] = 1.0
_WGT_NP[0, 1:_L // 2] = 2.0
_WGT_NP[0, _L // 2] = 1.0
del _ang, _n, _COS_NP, _SIN_NP

_HI = jax.lax.Precision.HIGHEST
_KT = 384
_TT = 512


def _mm(a, b):
    return jax.lax.dot_general(a, b, (((1,), (0,)), ((), ())),
                               precision=_HI, preferred_element_type=jnp.float32)


def _mmT(a, b):
    return jax.lax.dot_general(a, b, (((0,), (0,)), ((), ())),
                               precision=_HI, preferred_element_type=jnp.float32)


def _mmRT(a, b):
    return jax.lax.dot_general(a, b, (((1,), (1,)), ((), ())),
                               precision=_HI, preferred_element_type=jnp.float32)


def _proj_kernel(q_ref, k_ref, v_ref, wq_ref, bq_ref, oq_ref, ok_ref, ov_ref):
    t = q_ref.shape[1]
    x = jnp.concatenate([q_ref[0], k_ref[0], v_ref[0]], axis=0)
    y = jax.lax.dot_general(
        x.astype(jnp.bfloat16), wq_ref[...].astype(jnp.bfloat16),
        (((1,), (0,)), ((), ())),
        preferred_element_type=jnp.float32) + bq_ref[...]
    oq_ref[0] = y[:t]
    ok_ref[0] = y[t:2 * t]
    ov_ref[0] = y[2 * t:]


def _make_fftcorr_kernel(ktop, length, nb):
    inv_l = 1.0 / length

    def _fftcorr_kernel(q_ref, k_ref, v_ref, fc_ref, fs_ref, wgt_ref,
                        w_ref, vr_ref, vi_ref, acc_ref):
        j = pl.program_id(0)
        dh = q_ref.shape[2]
        prs, pis, vrs, vis = [], [], [], []
        for b in range(nb):
            x = jnp.concatenate([q_ref[b], k_ref[b], v_ref[b]], axis=1)
            xr = _mmT(x, fc_ref[...])
            xi = -_mmT(x, fs_ref[...])
            qr, kr, vr = xr[:dh], xr[dh:2 * dh], xr[2 * dh:]
            qi, ki, vi = xi[:dh], xi[dh:2 * dh], xi[2 * dh:]
            prs.append(qr * kr + qi * ki)
            pis.append(qi * kr - qr * ki)
            vrs.append(vr)
            vis.append(vi)
        pr = jnp.concatenate(prs, axis=0)
        pi = jnp.concatenate(pis, axis=0)
        vr_ref[...] = jnp.concatenate(vrs, axis=0)
        vi_ref[...] = jnp.concatenate(vis, axis=0)
        wgt = wgt_ref[...]
        contrib = _mmRT(pr * wgt, fc_ref[...]) - _mmRT(pi * wgt, fs_ref[...])

        @pl.when(j == 0)
        def _():
            acc_ref[...] = contrib

        @pl.when(j > 0)
        def _():
            acc_ref[...] += contrib

        @pl.when(j == pl.num_programs(0) - 1)
        def _():
            c = acc_ref[...] * inv_l
            iota = jax.lax.broadcasted_iota(jnp.int32, c.shape, 1)
            vals, idxs = [], []
            for _ in range(ktop):
                m = jnp.max(c, axis=1, keepdims=True)
                sel = jnp.where(c == m, iota, length)
                idx = jnp.min(sel, axis=1, keepdims=True)
                vals.append(m)
                idxs.append(idx)
                c = jnp.where(iota == idx, -jnp.inf, c)
            v0 = vals[0]
            es = [jnp.exp(v - v0) for v in vals]
            denom = es[0]
            for e in es[1:]:
                denom = denom + e
            w = jnp.zeros(c.shape, jnp.float32)
            for i in range(ktop):
                w = w + jnp.where(iota == idxs[i], es[i] / denom, 0.0)
            w_ref[...] = w

    return _fftcorr_kernel


def _make_agg_kernel(length, nb, heads):
    inv_l = 1.0 / length

    def _agg_kernel(w_ref, vr_ref, vi_ref, fc_ref, fs_ref, wgt_ref,
                    out_ref, acc_ref):
        j = pl.program_id(0)
        wc = _mm(w_ref[...], fc_ref[...])
        ws = _mm(w_ref[...], fs_ref[...])
        vr = vr_ref[...]
        vi = vi_ref[...]
        gr = vr * wc - vi * ws
        gi = vr * ws + vi * wc
        wgt = wgt_ref[...]
        contrib = (_mmRT(gr * wgt, fc_ref[...])
                   - _mmRT(gi * wgt, fs_ref[...])) * inv_l

        @pl.when(j == 0)
        def _():
            acc_ref[...] = contrib

        @pl.when(j > 0)
        def _():
            acc_ref[...] += contrib

        @pl.when(j == pl.num_programs(0) - 1)
        def _():
            agg = acc_ref[...]
            dh = agg.shape[0] // nb
            for b in range(nb):
                t = agg[b * dh:(b + 1) * dh].T
                out_ref[b] = jnp.concatenate([t] * heads, axis=-1)

    return _agg_kernel


def kernel(Q, K, V, Wq, bq):
    B, L, D = Q.shape
    dh = Wq.shape[1]
    heads = D // dh
    R = B * dh
    ktop = int(3 * math.log(L))
    assert L == _L, "DFT tables are built for L=2048"

    fc = jnp.asarray(_FC_NP)
    fs = jnp.asarray(_FS_NP)
    wgt = jnp.asarray(_WGT_NP)
    f32 = jnp.float32

    grid_a = (B, L // _TT)
    in_spec_x = pl.BlockSpec((1, _TT, D), lambda b, t: (b, t, 0))
    q, k, v = pl.pallas_call(
        _proj_kernel,
        grid=grid_a,
        in_specs=[in_spec_x, in_spec_x, in_spec_x,
                  pl.BlockSpec((D, dh), lambda b, t: (0, 0)),
                  pl.BlockSpec((1, dh), lambda b, t: (0, 0))],
        out_specs=[pl.BlockSpec((1, _TT, dh), lambda b, t: (b, t, 0))] * 3,
        out_shape=[jax.ShapeDtypeStruct((B, L, dh), f32)] * 3,
    )(Q, K, V, Wq, bq.reshape(1, dh))

    seq_spec = pl.BlockSpec((B, L, dh), lambda j: (0, 0, 0))
    fcol_spec = pl.BlockSpec((L, _KT), lambda j: (0, j))
    wgt_spec = pl.BlockSpec((1, _KT), lambda j: (0, j))
    spec_tile = pl.BlockSpec((R, _KT), lambda j: (0, j))
    full_spec = pl.BlockSpec((R, L), lambda j: (0, 0))
    w, vr, vi = pl.pallas_call(
        _make_fftcorr_kernel(ktop, L, B),
        grid=(_KH // _KT,),
        in_specs=[seq_spec, seq_spec, seq_spec,
                  fcol_spec, fcol_spec, wgt_spec],
        out_specs=[full_spec, spec_tile, spec_tile],
        out_shape=[jax.ShapeDtypeStruct((R, L), f32),
                   jax.ShapeDtypeStruct((R, _KH), f32),
                   jax.ShapeDtypeStruct((R, _KH), f32)],
        scratch_shapes=[pltpu.VMEM((R, L), f32)],
    )(q, k, v, fc, fs, wgt)

    out = pl.pallas_call(
        _make_agg_kernel(L, B, heads),
        grid=(_KH // _KT,),
        in_specs=[full_spec, spec_tile, spec_tile,
                  fcol_spec, fcol_spec, wgt_spec],
        out_specs=pl.BlockSpec((B, L, D), lambda j: (0, 0, 0)),
        out_shape=jax.ShapeDtypeStruct((B, L, D), f32),
        scratch_shapes=[pltpu.VMEM((R, L), f32)],
    )(w, vr, vi, fc, fs, wgt)
    return out


# speedup vs baseline: 1.0830x; 1.0830x over previous
import math

import numpy as np
import jax
import jax.numpy as jnp
from jax.experimental import pallas as pl
from jax.experimental.pallas import tpu as pltpu

_L = 2048
_KH = 1152
_n = np.arange(_L, dtype=np.int64)
_ang = (2.0 * np.pi / _L) * (np.outer(_n, _n) % _L)
_COS_NP = np.cos(_ang).astype(np.float32)
_SIN_NP = np.sin(_ang).astype(np.float32)
_FC_NP = _COS_NP[:, :_KH].copy()
_FS_NP = _SIN_NP[:, :_KH].copy()
_wgt = np.zeros((_KH, 1), np.float32)
_wgt[0] = 1.0
_wgt[1:_L // 2] = 2.0
_wgt[_L // 2] = 1.0
_IC_NP = (_wgt * _COS_NP[:_KH, :]).astype(np.float32)
_IS_NP = (_wgt * _SIN_NP[:_KH, :]).astype(np.float32)
del _ang, _n, _COS_NP, _SIN_NP, _wgt

_HI = jax.lax.Precision.HIGHEST
_KT = 384
_TT = 1024


def _mm(a, b):
    return jax.lax.dot_general(a, b, (((1,), (0,)), ((), ())),
                               precision=_HI, preferred_element_type=jnp.float32)


def _mmT(a, b):
    return jax.lax.dot_general(a, b, (((0,), (0,)), ((), ())),
                               precision=_HI, preferred_element_type=jnp.float32)


def _proj_kernel(q_ref, k_ref, v_ref, wq_ref, bq_ref, oq_ref, ok_ref, ov_ref):
    t = q_ref.shape[1]
    x = jnp.concatenate([q_ref[0], k_ref[0], v_ref[0]], axis=0)
    y = jax.lax.dot_general(
        x.astype(jnp.bfloat16), wq_ref[...].astype(jnp.bfloat16),
        (((1,), (0,)), ((), ())),
        preferred_element_type=jnp.float32) + bq_ref[...]
    oq_ref[0] = y[:t]
    ok_ref[0] = y[t:2 * t]
    ov_ref[0] = y[2 * t:]


def _make_fftcorr_kernel(ktop, length, nb):
    inv_l = 1.0 / length

    def _fftcorr_kernel(q_ref, k_ref, v_ref, fc_ref, fs_ref, ic_ref, is_ref,
                        w_ref, vr_ref, vi_ref, acc_ref):
        j = pl.program_id(0)
        dh = q_ref.shape[2]
        prs, pis, vrs, vis = [], [], [], []
        for b in range(nb):
            x = jnp.concatenate([q_ref[b], k_ref[b], v_ref[b]], axis=1)
            xr = _mmT(x, fc_ref[...])
            xi = -_mmT(x, fs_ref[...])
            qr, kr, vr = xr[:dh], xr[dh:2 * dh], xr[2 * dh:]
            qi, ki, vi = xi[:dh], xi[dh:2 * dh], xi[2 * dh:]
            prs.append(qr * kr + qi * ki)
            pis.append(qi * kr - qr * ki)
            vrs.append(vr)
            vis.append(vi)
        pr = jnp.concatenate(prs, axis=0)
        pi = jnp.concatenate(pis, axis=0)
        vr_ref[...] = jnp.concatenate(vrs, axis=0)
        vi_ref[...] = jnp.concatenate(vis, axis=0)
        contrib = _mm(pr, ic_ref[...]) - _mm(pi, is_ref[...])

        @pl.when(j == 0)
        def _():
            acc_ref[...] = contrib

        @pl.when(j > 0)
        def _():
            acc_ref[...] += contrib

        @pl.when(j == pl.num_programs(0) - 1)
        def _():
            c = acc_ref[...] * inv_l
            iota = jax.lax.broadcasted_iota(jnp.int32, c.shape, 1)
            vals, idxs = [], []
            for _ in range(ktop):
                m = jnp.max(c, axis=1, keepdims=True)
                sel = jnp.where(c == m, iota, length)
                idx = jnp.min(sel, axis=1, keepdims=True)
                vals.append(m)
                idxs.append(idx)
                c = jnp.where(iota == idx, -jnp.inf, c)
            v0 = vals[0]
            es = [jnp.exp(v - v0) for v in vals]
            denom = es[0]
            for e in es[1:]:
                denom = denom + e
            w = jnp.zeros(c.shape, jnp.float32)
            for i in range(ktop):
                w = w + jnp.where(iota == idxs[i], es[i] / denom, 0.0)
            w_ref[...] = w

    return _fftcorr_kernel


def _make_agg_kernel(length, nb, heads):
    inv_l = 1.0 / length

    def _agg_kernel(w_ref, vr_ref, vi_ref, fc_ref, fs_ref, ic_ref, is_ref,
                    out_ref, acc_ref):
        j = pl.program_id(0)
        wc = _mm(w_ref[...], fc_ref[...])
        ws = _mm(w_ref[...], fs_ref[...])
        vr = vr_ref[...]
        vi = vi_ref[...]
        gr = vr * wc - vi * ws
        gi = vr * ws + vi * wc
        contrib = (_mm(gr, ic_ref[...]) - _mm(gi, is_ref[...])) * inv_l

        @pl.when(j == 0)
        def _():
            acc_ref[...] = contrib

        @pl.when(j > 0)
        def _():
            acc_ref[...] += contrib

        @pl.when(j == pl.num_programs(0) - 1)
        def _():
            agg = acc_ref[...]
            dh = agg.shape[0] // nb
            for b in range(nb):
                t = agg[b * dh:(b + 1) * dh].T
                out_ref[b] = jnp.concatenate([t] * heads, axis=-1)

    return _agg_kernel


def kernel(Q, K, V, Wq, bq):
    B, L, D = Q.shape
    dh = Wq.shape[1]
    heads = D // dh
    R = B * dh
    ktop = int(3 * math.log(L))
    assert L == _L, "DFT tables are built for L=2048"

    fc = jnp.asarray(_FC_NP)
    fs = jnp.asarray(_FS_NP)
    ic = jnp.asarray(_IC_NP)
    is_ = jnp.asarray(_IS_NP)
    f32 = jnp.float32

    grid_a = (B, L // _TT)
    in_spec_x = pl.BlockSpec((1, _TT, D), lambda b, t: (b, t, 0))
    q, k, v = pl.pallas_call(
        _proj_kernel,
        grid=grid_a,
        in_specs=[in_spec_x, in_spec_x, in_spec_x,
                  pl.BlockSpec((D, dh), lambda b, t: (0, 0)),
                  pl.BlockSpec((1, dh), lambda b, t: (0, 0))],
        out_specs=[pl.BlockSpec((1, _TT, dh), lambda b, t: (b, t, 0))] * 3,
        out_shape=[jax.ShapeDtypeStruct((B, L, dh), f32)] * 3,
    )(Q, K, V, Wq, bq.reshape(1, dh))

    seq_spec = pl.BlockSpec((B, L, dh), lambda j: (0, 0, 0))
    fcol_spec = pl.BlockSpec((L, _KT), lambda j: (0, j))
    irow_spec = pl.BlockSpec((_KT, L), lambda j: (j, 0))
    spec_tile = pl.BlockSpec((R, _KT), lambda j: (0, j))
    full_spec = pl.BlockSpec((R, L), lambda j: (0, 0))
    w, vr, vi = pl.pallas_call(
        _make_fftcorr_kernel(ktop, L, B),
        grid=(_KH // _KT,),
        in_specs=[seq_spec, seq_spec, seq_spec,
                  fcol_spec, fcol_spec, irow_spec, irow_spec],
        out_specs=[full_spec, spec_tile, spec_tile],
        out_shape=[jax.ShapeDtypeStruct((R, L), f32),
                   jax.ShapeDtypeStruct((R, _KH), f32),
                   jax.ShapeDtypeStruct((R, _KH), f32)],
        scratch_shapes=[pltpu.VMEM((R, L), f32)],
    )(q, k, v, fc, fs, ic, is_)

    out = pl.pallas_call(
        _make_agg_kernel(L, B, heads),
        grid=(_KH // _KT,),
        in_specs=[full_spec, spec_tile, spec_tile,
                  fcol_spec, fcol_spec, irow_spec, irow_spec],
        out_specs=pl.BlockSpec((B, L, D), lambda j: (0, 0, 0)),
        out_shape=jax.ShapeDtypeStruct((B, L, D), f32),
        scratch_shapes=[pltpu.VMEM((R, L), f32)],
    )(w, vr, vi, fc, fs, ic, is_)
    return out
